# baseline (speedup 1.0000x reference)
import jax
import jax.numpy as jnp
from jax.experimental import pallas as pl
from jax.experimental.pallas import tpu as pltpu

T = 2048
D = 2048
E = 8
TOPK = 2
I_MOE = 1024
I_SHARED = 2048

TB = 256
SB = 128
RB = 128
NTB = T // TB
NSB = T // SB
NA = T * TOPK
R_PAD = 4992
NB = R_PAD // RB


def _router_body(x_ref, gate_ref, ebias_ref, route_ref):
    x = x_ref[...]
    logits = jax.lax.dot_general(
        x, gate_ref[...], (((1,), (1,)), ((), ())),
        preferred_element_type=jnp.float32)
    scores = jax.nn.sigmoid(logits)
    sfc = scores + ebias_ref[...]
    lane = jax.lax.broadcasted_iota(jnp.int32, (TB, E), 1)
    big = jnp.float32(1e30)
    m1 = jnp.max(sfc, axis=1, keepdims=True)
    i1 = jnp.min(jnp.where(sfc == m1, lane, E), axis=1, keepdims=True)
    oh1 = lane == i1
    sfc2 = jnp.where(oh1, -big, sfc)
    m2 = jnp.max(sfc2, axis=1, keepdims=True)
    i2 = jnp.min(jnp.where(sfc2 == m2, lane, E), axis=1, keepdims=True)
    oh2 = lane == i2
    w1 = jnp.sum(jnp.where(oh1, scores, 0.0), axis=1, keepdims=True)
    w2 = jnp.sum(jnp.where(oh2, scores, 0.0), axis=1, keepdims=True)
    denom = w1 + w2
    route_ref[...] = jnp.where(
        lane == 0, i1.astype(jnp.float32),
        jnp.where(lane == 1, i2.astype(jnp.float32),
                  jnp.where(lane == 2, w1 / denom,
                            jnp.where(lane == 3, w2 / denom, 0.0))))


def _shared_body(x_ref, sgu_ref, sdn_ref, shared_ref):
    gu = jax.lax.dot_general(
        x_ref[...], sgu_ref[...], (((1,), (1,)), ((), ())),
        preferred_element_type=jnp.float32)
    a = gu[:, :I_SHARED]
    b = gu[:, I_SHARED:]
    h = a * jax.nn.sigmoid(a) * b
    shared_ref[...] = jax.lax.dot_general(
        h, sdn_ref[...], (((1,), (1,)), ((), ())),
        preferred_element_type=jnp.float32)


def _grouped_body(be_ref, x_ref, wgu_ref, wdn_ref, y_ref):
    gu = jax.lax.dot_general(
        x_ref[...], wgu_ref[0], (((1,), (1,)), ((), ())),
        preferred_element_type=jnp.float32)
    a = gu[:, :I_MOE]
    b = gu[:, I_MOE:]
    h = a * jax.nn.sigmoid(a) * b
    y_ref[...] = jax.lax.dot_general(
        h, wdn_ref[0], (((1,), (1,)), ((), ())),
        preferred_element_type=jnp.float32)


def kernel(hidden_states, gate_w, e_bias, w_gate_up, w_down, s_gate_up, s_down):
    x = hidden_states

    route = pl.pallas_call(
        _router_body,
        grid=(NTB,),
        in_specs=[
            pl.BlockSpec((TB, D), lambda i: (i, 0)),
            pl.BlockSpec((E, D), lambda i: (0, 0)),
            pl.BlockSpec((1, E), lambda i: (0, 0)),
        ],
        out_specs=pl.BlockSpec((TB, E), lambda i: (i, 0)),
        out_shape=jax.ShapeDtypeStruct((T, E), jnp.float32),
        compiler_params=pltpu.CompilerParams(
            dimension_semantics=("arbitrary",)),
    )(x, gate_w, e_bias.reshape(1, E))

    shared = pl.pallas_call(
        _shared_body,
        grid=(NSB,),
        in_specs=[
            pl.BlockSpec((SB, D), lambda i: (i, 0)),
            pl.BlockSpec((2 * I_SHARED, D), lambda i: (0, 0)),
            pl.BlockSpec((D, I_SHARED), lambda i: (0, 0)),
        ],
        out_specs=pl.BlockSpec((SB, D), lambda i: (i, 0)),
        out_shape=jax.ShapeDtypeStruct((T, D), jnp.float32),
        compiler_params=pltpu.CompilerParams(
            dimension_semantics=("arbitrary",)),
    )(x, s_gate_up, s_down)

    topk_idx = route[:, :TOPK].astype(jnp.int32)
    topk_w = route[:, TOPK:2 * TOPK]

    ids = topk_idx.reshape(-1)
    order = jnp.argsort(ids, stable=True).astype(jnp.int32)
    ids_sorted = ids[order]
    counts = jnp.zeros((E,), jnp.int32).at[ids].add(1)
    padded = ((counts + RB - 1) // RB) * RB
    pstart = jnp.concatenate([jnp.zeros((1,), jnp.int32),
                              jnp.cumsum(padded)])[:E]
    cstart = jnp.concatenate([jnp.zeros((1,), jnp.int32),
                              jnp.cumsum(counts)])[:E]
    rank = jnp.arange(NA, dtype=jnp.int32) - cstart[ids_sorted]
    dest_sorted = pstart[ids_sorted] + rank
    srctid = jnp.zeros((R_PAD,), jnp.int32).at[dest_sorted].set(order // TOPK)
    dpos = jnp.zeros((NA,), jnp.int32).at[order].set(dest_sorted)
    dpos = dpos.reshape(T, TOPK)
    ends = jnp.cumsum(padded)
    block_expert = jnp.minimum(
        jnp.searchsorted(ends, jnp.arange(NB, dtype=jnp.int32) * RB,
                         side="right").astype(jnp.int32), E - 1)

    x_sorted = jnp.take(x, srctid, axis=0)

    y = pl.pallas_call(
        _grouped_body,
        grid_spec=pltpu.PrefetchScalarGridSpec(
            num_scalar_prefetch=1,
            grid=(NB,),
            in_specs=[
                pl.BlockSpec((RB, D), lambda b, be: (b, 0)),
                pl.BlockSpec((1, 2 * I_MOE, D), lambda b, be: (be[b], 0, 0)),
                pl.BlockSpec((1, D, I_MOE), lambda b, be: (be[b], 0, 0)),
            ],
            out_specs=pl.BlockSpec((RB, D), lambda b, be: (b, 0)),
        ),
        out_shape=jax.ShapeDtypeStruct((R_PAD, D), jnp.float32),
        compiler_params=pltpu.CompilerParams(
            dimension_semantics=("arbitrary",)),
    )(block_expert, x_sorted, w_gate_up, w_down)

    y0 = jnp.take(y, dpos[:, 0], axis=0)
    y1 = jnp.take(y, dpos[:, 1], axis=0)
    return shared + topk_w[:, :1] * y0 + topk_w[:, 1:] * y1


# speedup vs baseline: 1.1775x; 1.1775x over previous
import functools

import jax
import jax.numpy as jnp
from jax import lax
from jax.experimental import pallas as pl
from jax.experimental.pallas import tpu as pltpu
from jax.experimental.pallas import tpu_sc as plsc

T = 2048
D = 2048
E = 8
TOPK = 2
I_MOE = 1024
I_SHARED = 2048

TB = 256
SB = 256
RB = 256
NTB = T // TB
NSB = T // SB
NA = T * TOPK
R_PAD = 5888
NB = R_PAD // RB
SRC_PAD = 6144

SC_CORES = 2
SC_SUBCORES = 16
NW = SC_CORES * SC_SUBCORES
XG_PER_W = SRC_PAD // NW
YG_PER_W = NA // NW


def _router_body(x_ref, gate_ref, ebias_ref, route_ref, xb_ref):
    x = x_ref[...]
    xb_ref[...] = x.astype(jnp.bfloat16)
    logits = jax.lax.dot_general(
        x, gate_ref[...], (((1,), (1,)), ((), ())),
        preferred_element_type=jnp.float32)
    scores = jax.nn.sigmoid(logits)
    sfc = scores + ebias_ref[...]
    lane = jax.lax.broadcasted_iota(jnp.int32, (TB, E), 1)
    big = jnp.float32(1e30)
    m1 = jnp.max(sfc, axis=1, keepdims=True)
    i1 = jnp.min(jnp.where(sfc == m1, lane, E), axis=1, keepdims=True)
    oh1 = lane == i1
    sfc2 = jnp.where(oh1, -big, sfc)
    m2 = jnp.max(sfc2, axis=1, keepdims=True)
    i2 = jnp.min(jnp.where(sfc2 == m2, lane, E), axis=1, keepdims=True)
    oh2 = lane == i2
    w1 = jnp.sum(jnp.where(oh1, scores, 0.0), axis=1, keepdims=True)
    w2 = jnp.sum(jnp.where(oh2, scores, 0.0), axis=1, keepdims=True)
    denom = w1 + w2
    route_ref[...] = jnp.where(
        lane == 0, i1.astype(jnp.float32),
        jnp.where(lane == 1, i2.astype(jnp.float32),
                  jnp.where(lane == 2, w1 / denom,
                            jnp.where(lane == 3, w2 / denom, 0.0))))


def _shared_gu_body(xb_ref, sgu_ref, h_ref, wbf_ref):
    @pl.when(pl.program_id(0) == 0)
    def _cast():
        wbf_ref[...] = sgu_ref[...].astype(jnp.bfloat16)

    gu = jax.lax.dot_general(
        xb_ref[...], wbf_ref[...], (((1,), (1,)), ((), ())),
        preferred_element_type=jnp.float32)
    a = gu[:, :I_SHARED]
    b = gu[:, I_SHARED:]
    h_ref[...] = (a * jax.nn.sigmoid(a) * b).astype(jnp.bfloat16)


def _shared_dn_body(h_ref, sdn_ref, shared_ref, wbf_ref):
    @pl.when(pl.program_id(0) == 0)
    def _cast():
        wbf_ref[...] = sdn_ref[...].astype(jnp.bfloat16)

    shared_ref[...] = jax.lax.dot_general(
        h_ref[...], wbf_ref[...], (((1,), (1,)), ((), ())),
        preferred_element_type=jnp.float32)


def _gate_up_body(be_ref, x_ref, wgu_ref, dep_ref, h_ref, wbf_ref):
    del dep_ref
    b = pl.program_id(0)

    @pl.when((b == 0) | (be_ref[b] != be_ref[jnp.maximum(b - 1, 0)]))
    def _cast():
        wbf_ref[...] = wgu_ref[0].astype(jnp.bfloat16)

    gu = jax.lax.dot_general(
        x_ref[...].astype(jnp.bfloat16), wbf_ref[...],
        (((1,), (1,)), ((), ())),
        preferred_element_type=jnp.float32)
    a = gu[:, :I_MOE]
    bb = gu[:, I_MOE:]
    h_ref[...] = (a * jax.nn.sigmoid(a) * bb).astype(jnp.bfloat16)


def _down_body(be_ref, h_ref, wdn_ref, w_ref, y_ref, wbf_ref):
    b = pl.program_id(0)

    @pl.when((b == 0) | (be_ref[b] != be_ref[jnp.maximum(b - 1, 0)]))
    def _cast():
        wbf_ref[...] = wdn_ref[0].astype(jnp.bfloat16)

    y = jax.lax.dot_general(
        h_ref[...], wbf_ref[...], (((1,), (1,)), ((), ())),
        preferred_element_type=jnp.float32)
    y_ref[...] = y * w_ref[:, :1]


def _combine_body(sh_ref, y0_ref, y1_ref, out_ref):
    out_ref[...] = sh_ref[...] + y0_ref[...] + y1_ref[...]


@functools.cache
def _make_sc_gather(n_rows, per_w, chunk, width, dtype):
    n_chunks = per_w // chunk
    nbuf = 3

    @functools.partial(
        pl.kernel,
        mesh=plsc.VectorSubcoreMesh(core_axis_name="c", subcore_axis_name="s"),
        out_type=jax.ShapeDtypeStruct((n_rows, width), dtype),
        scratch_types=[
            pltpu.VMEM((per_w,), jnp.int32),
            pltpu.VMEM((chunk, width), dtype),
            pltpu.VMEM((chunk, width), dtype),
            pltpu.VMEM((chunk, width), dtype),
            pltpu.SemaphoreType.DMA,
            pltpu.SemaphoreType.DMA,
            pltpu.SemaphoreType.DMA,
            pltpu.SemaphoreType.DMA,
            pltpu.SemaphoreType.DMA,
            pltpu.SemaphoreType.DMA,
        ],
    )
    def gather(src_hbm, idx_hbm, out_hbm, idx_v, b0, b1, b2,
               g0, g1, g2, s0, s1, s2):
        wid = lax.axis_index("s") * SC_CORES + lax.axis_index("c")
        base = wid * per_w
        pltpu.sync_copy(idx_hbm.at[pl.ds(base, per_w)], idx_v)
        bufs = (b0, b1, b2)
        gsems = (g0, g1, g2)
        ssems = (s0, s1, s2)
        gcp = [None] * n_chunks
        scp = [None] * n_chunks

        def start_gather(c):
            gcp[c] = pltpu.async_copy(
                src_hbm.at[idx_v.at[pl.ds(c * chunk, chunk)]],
                bufs[c % nbuf], gsems[c % nbuf])

        for c in range(min(nbuf - 1, n_chunks)):
            start_gather(c)
        for c in range(n_chunks):
            gcp[c].wait()
            scp[c] = pltpu.async_copy(
                bufs[c % nbuf],
                out_hbm.at[pl.ds(base + c * chunk, chunk)],
                ssems[c % nbuf])
            nxt = c + nbuf - 1
            if nxt < n_chunks:
                prev = nxt - nbuf
                if prev >= 0:
                    scp[prev].wait()
                start_gather(nxt)
        for c in range(max(0, n_chunks - nbuf), n_chunks):
            if scp[c] is not None:
                scp[c].wait()
    return gather


def _sc_xgather(src, idx):
    return _make_sc_gather(SRC_PAD, XG_PER_W, 16, D, jnp.float32)(src, idx)


def _sc_ygather(src, idx):
    return _make_sc_gather(NA, YG_PER_W, 16, D, jnp.float32)(src, idx)


def kernel(hidden_states, gate_w, e_bias, w_gate_up, w_down, s_gate_up, s_down):
    x = hidden_states

    route, xb = pl.pallas_call(
        _router_body,
        grid=(NTB,),
        in_specs=[
            pl.BlockSpec((TB, D), lambda i: (i, 0)),
            pl.BlockSpec((E, D), lambda i: (0, 0)),
            pl.BlockSpec((1, E), lambda i: (0, 0)),
        ],
        out_specs=[
            pl.BlockSpec((TB, E), lambda i: (i, 0)),
            pl.BlockSpec((TB, D), lambda i: (i, 0)),
        ],
        out_shape=[
            jax.ShapeDtypeStruct((T, E), jnp.float32),
            jax.ShapeDtypeStruct((T, D), jnp.bfloat16),
        ],
        compiler_params=pltpu.CompilerParams(
            dimension_semantics=("arbitrary",)),
    )(x, gate_w, e_bias.reshape(1, E))

    sh = pl.pallas_call(
        _shared_gu_body,
        grid=(NSB,),
        in_specs=[
            pl.BlockSpec((SB, D), lambda i: (i, 0)),
            pl.BlockSpec((2 * I_SHARED, D), lambda i: (0, 0)),
        ],
        out_specs=pl.BlockSpec((SB, I_SHARED), lambda i: (i, 0)),
        out_shape=jax.ShapeDtypeStruct((T, I_SHARED), jnp.bfloat16),
        scratch_shapes=[pltpu.VMEM((2 * I_SHARED, D), jnp.bfloat16)],
        compiler_params=pltpu.CompilerParams(
            dimension_semantics=("arbitrary",)),
    )(xb, s_gate_up)

    shared = pl.pallas_call(
        _shared_dn_body,
        grid=(NSB,),
        in_specs=[
            pl.BlockSpec((SB, I_SHARED), lambda i: (i, 0)),
            pl.BlockSpec((D, I_SHARED), lambda i: (0, 0)),
        ],
        out_specs=pl.BlockSpec((SB, D), lambda i: (i, 0)),
        out_shape=jax.ShapeDtypeStruct((T, D), jnp.float32),
        scratch_shapes=[pltpu.VMEM((D, I_SHARED), jnp.bfloat16)],
        compiler_params=pltpu.CompilerParams(
            dimension_semantics=("arbitrary",)),
    )(sh, s_down)

    topk_idx = route[:, :TOPK].astype(jnp.int32)
    topk_w = route[:, TOPK:2 * TOPK]

    ids = topk_idx.reshape(-1)
    order = jnp.argsort(ids, stable=True).astype(jnp.int32)
    ids_sorted = ids[order]
    counts = jnp.zeros((E,), jnp.int32).at[ids].add(1)
    padded = ((counts + RB - 1) // RB) * RB
    pstart = jnp.concatenate([jnp.zeros((1,), jnp.int32),
                              jnp.cumsum(padded)])[:E]
    cstart = jnp.concatenate([jnp.zeros((1,), jnp.int32),
                              jnp.cumsum(counts)])[:E]
    rank = jnp.arange(NA, dtype=jnp.int32) - cstart[ids_sorted]
    dest_sorted = pstart[ids_sorted] + rank
    packed_val = jnp.stack(
        [order // TOPK,
         jax.lax.bitcast_convert_type(topk_w.reshape(-1)[order], jnp.int32)],
        axis=1)
    packed = jnp.zeros((SRC_PAD, 2), jnp.int32).at[dest_sorted].set(packed_val)
    srctid = packed[:, 0]
    w_sorted = jax.lax.bitcast_convert_type(packed[:R_PAD, 1], jnp.float32)
    dpos = jnp.zeros((NA,), jnp.int32).at[order].set(dest_sorted)
    dpos = dpos.reshape(T, TOPK)
    d01 = jnp.concatenate([dpos[:, 0], dpos[:, 1]])
    ends = jnp.cumsum(padded)
    block_expert = jnp.minimum(
        jnp.searchsorted(ends, jnp.arange(NB, dtype=jnp.int32) * RB,
                         side="right").astype(jnp.int32), E - 1)
    w_bcast = jnp.broadcast_to(w_sorted[:, None], (R_PAD, 128))

    x_sorted = _sc_xgather(x, srctid)

    h = pl.pallas_call(
        _gate_up_body,
        grid_spec=pltpu.PrefetchScalarGridSpec(
            num_scalar_prefetch=1,
            grid=(NB,),
            in_specs=[
                pl.BlockSpec((RB, D), lambda b, be: (b, 0)),
                pl.BlockSpec((1, 2 * I_MOE, D), lambda b, be: (be[b], 0, 0)),
                pl.BlockSpec((8, 128), lambda b, be: (0, 0)),
            ],
            out_specs=pl.BlockSpec((RB, I_MOE), lambda b, be: (b, 0)),
            scratch_shapes=[pltpu.VMEM((2 * I_MOE, D), jnp.bfloat16)],
        ),
        out_shape=jax.ShapeDtypeStruct((R_PAD, I_MOE), jnp.bfloat16),
        compiler_params=pltpu.CompilerParams(
            dimension_semantics=("arbitrary",)),
    )(block_expert, x_sorted, w_gate_up, shared)

    y = pl.pallas_call(
        _down_body,
        grid_spec=pltpu.PrefetchScalarGridSpec(
            num_scalar_prefetch=1,
            grid=(NB,),
            in_specs=[
                pl.BlockSpec((RB, I_MOE), lambda b, be: (b, 0)),
                pl.BlockSpec((1, D, I_MOE), lambda b, be: (be[b], 0, 0)),
                pl.BlockSpec((RB, 128), lambda b, be: (b, 0)),
            ],
            out_specs=pl.BlockSpec((RB, D), lambda b, be: (b, 0)),
            scratch_shapes=[pltpu.VMEM((D, I_MOE), jnp.bfloat16)],
        ),
        out_shape=jax.ShapeDtypeStruct((R_PAD, D), jnp.float32),
        compiler_params=pltpu.CompilerParams(
            dimension_semantics=("arbitrary",)),
    )(block_expert, h, w_down, w_bcast)

    yg = _sc_ygather(y, d01)

    return pl.pallas_call(
        _combine_body,
        grid=(NTB,),
        in_specs=[
            pl.BlockSpec((TB, D), lambda i: (i, 0)),
            pl.BlockSpec((TB, D), lambda i: (i, 0)),
            pl.BlockSpec((TB, D), lambda i: (i + T // TB, 0)),
        ],
        out_specs=pl.BlockSpec((TB, D), lambda i: (i, 0)),
        out_shape=jax.ShapeDtypeStruct((T, D), jnp.float32),
        compiler_params=pltpu.CompilerParams(
            dimension_semantics=("arbitrary",)),
    )(shared, yg, yg)
